# Initial kernel scaffold; baseline (speedup 1.0000x reference)
import jax
import jax.numpy as jnp
from jax.experimental import pallas as pl


def kernel(x, edge_index, edge_vals, log_alpha_drug, log_alpha_prot, log_t_drug, log_t_prot):
    raise NotImplementedError("write your pallas kernel here")


# speedup vs baseline: 1.8850x; 1.8850x over previous
import functools

import jax
import jax.numpy as jnp
from jax import lax
from jax.experimental import pallas as pl
from jax.experimental.pallas import tpu as pltpu
from jax.experimental.pallas import tpu_sc as plsc

_K = 15
_DRUG = 5000
_N = 10000
_D = 128
_E = 320000

_NC = 2
_NS = 16
_FH = _D // _NC
_FV = _FH // 16
_RT = _N // _NS
_ZR = 25
_G = 64
_EPT = -(-_E // (_NS * _G)) * _G
_C = _EPT // _G
_EP = _EPT * _NS
_FR = _RT // _G
_FREM = _RT - _FR * _G


def _frac_coeffs(log_alpha, log_t):
    alpha = jnp.clip(jnp.logaddexp(log_alpha, 0.0), 0.05, 3.0)
    t = jnp.clip(jnp.logaddexp(log_t, 0.0), 0.01, 10.0)
    s = t / (1.0 + t)
    base = (1.0 + t) ** (-alpha)
    coeffs = [base]
    rising = jnp.ones_like(alpha)
    s_pow = jnp.ones_like(alpha)
    for k in range(1, _K + 1):
        rising = rising * (alpha + k - 1) / k
        s_pow = s_pow * s
        coeffs.append(base * rising * s_pow)
    return jnp.stack(coeffs)


def _body(x_hbm, e_hbm, v_hbm, coeff_hbm, out_hbm,
          P, Q, ebuf, vbuf, gbuf, out_t, zbuf, coeff_t):
    c = lax.axis_index("c")
    s = lax.axis_index("s")
    fbase = c * _FH
    rbase = s * _RT

    pltpu.sync_copy(coeff_hbm, coeff_t)
    pltpu.sync_copy(x_hbm.at[pl.ds(rbase, _RT), pl.ds(fbase, _FH)],
                    P.at[pl.ds(rbase, _RT)])

    zero16 = jnp.zeros((16,), jnp.float32)

    def _zrow(r, _):
        for f in range(_FV):
            zbuf[r, pl.ds(f * 16, 16)] = zero16
        return 0

    lax.fori_loop(0, _ZR, _zrow, 0)

    def _zout(r, _):
        for f in range(_FV):
            out_t[r, pl.ds(f * 16, 16)] = zero16
        return 0

    lax.fori_loop(0, _RT, _zout, 0)
    for j in range(_RT // _ZR):
        pltpu.sync_copy(zbuf, Q.at[pl.ds(rbase + j * _ZR, _ZR)])
    plsc.subcore_barrier()

    crow = jnp.where(s < _DRUG // _RT, 0, 1)

    def fma_rows(src_sh, base, rows, obase, ckv):
        pltpu.sync_copy(src_sh.at[pl.ds(base, rows)],
                        gbuf.at[pl.ds(0, rows)])

        def rloop(r, _):
            for f in range(_FV):
                sl = pl.ds(f * 16, 16)
                out_t[obase + r, sl] = out_t[obase + r, sl] + ckv * gbuf[r, sl]
            return 0

        lax.fori_loop(0, rows, rloop, 0)

    def fma_round(src_sh, kidx):
        ckv = coeff_t[crow, kidx]

        def chunk(j, _):
            fma_rows(src_sh, rbase + j * _G, _G, j * _G, ckv)
            return 0

        lax.fori_loop(0, _FR, chunk, 0)
        fma_rows(src_sh, rbase + _FR * _G, _FREM, _FR * _G, ckv)

    def spmm_round(A, B):
        def echunk(j, _):
            pltpu.sync_copy(e_hbm.at[s, j], ebuf)
            pltpu.sync_copy(v_hbm.at[s, j], vbuf)
            pltpu.sync_copy(A.at[ebuf.at[0]], gbuf)

            def scale16(g, _):
                vvec = vbuf[pl.ds(g * 16, 16)]
                for u in range(16):
                    e = g * 16 + u
                    vv = jnp.full((16,), vvec[u], jnp.float32)
                    for f in range(_FV):
                        sl = pl.ds(f * 16, 16)
                        gbuf[e, sl] = gbuf[e, sl] * vv
                return 0

            lax.fori_loop(0, _G // 16, scale16, 0)
            pltpu.sync_copy(gbuf, B.at[ebuf.at[1]], add=True)
            return 0

        lax.fori_loop(0, _C, echunk, 0)

    def finish_round(A):
        plsc.subcore_barrier()
        for j in range(_RT // _ZR):
            pltpu.sync_copy(zbuf, A.at[pl.ds(rbase + j * _ZR, _ZR)])
        plsc.subcore_barrier()

    def pair(i, _):
        fma_round(P, 2 * i)
        spmm_round(P, Q)
        finish_round(P)
        fma_round(Q, 2 * i + 1)
        spmm_round(Q, P)
        finish_round(Q)
        return 0

    lax.fori_loop(0, (_K - 1) // 2, pair, 0)
    fma_round(P, jnp.int32(_K - 1))
    spmm_round(P, Q)
    plsc.subcore_barrier()
    fma_round(Q, jnp.int32(_K))

    pltpu.sync_copy(out_t,
                    out_hbm.at[pl.ds(rbase, _RT), pl.ds(fbase, _FH)])


def kernel(x, edge_index, edge_vals, log_alpha_drug, log_alpha_prot,
           log_t_drug, log_t_prot):
    coeffs = jnp.stack([
        _frac_coeffs(log_alpha_drug, log_t_drug),
        _frac_coeffs(log_alpha_prot, log_t_prot),
    ]).astype(jnp.float32)
    coeffs = jnp.tile(coeffs[:, :, None], (1, 1, 16))

    dst = edge_index[0].astype(jnp.int32)
    src = edge_index[1].astype(jnp.int32)
    vals = edge_vals.astype(jnp.float32)
    pad = _EP - _E
    zi = jnp.zeros((pad,), jnp.int32)
    srcp = jnp.concatenate([src, zi]).reshape(_NS, _C, _G)
    dstp = jnp.concatenate([dst, zi]).reshape(_NS, _C, _G)
    valsp = jnp.concatenate([vals, jnp.zeros((pad,), jnp.float32)]
                            ).reshape(_NS, _C, _G)
    edges = jnp.stack([srcp, dstp], axis=2)

    mesh = plsc.VectorSubcoreMesh(core_axis_name="c", subcore_axis_name="s",
                                  num_cores=_NC, num_subcores=_NS)
    run = functools.partial(
        pl.kernel,
        out_type=jax.ShapeDtypeStruct((_N, _D), jnp.float32),
        mesh=mesh,
        compiler_params=pltpu.CompilerParams(use_tc_tiling_on_sc=False),
        scratch_types=[
            pltpu.VMEM_SHARED((_N, _FH), jnp.float32),
            pltpu.VMEM_SHARED((_N, _FH), jnp.float32),
            pltpu.VMEM((2, _G), jnp.int32),
            pltpu.VMEM((_G,), jnp.float32),
            pltpu.VMEM((_G, _FH), jnp.float32),
            pltpu.VMEM((_RT, _FH), jnp.float32),
            pltpu.VMEM((_ZR, _FH), jnp.float32),
            pltpu.VMEM((2, 16, 16), jnp.float32),
        ],
    )(_body)
    return run(x, edges, valsp, coeffs)


# speedup vs baseline: 3.4498x; 1.8301x over previous
import functools

import jax
import jax.numpy as jnp
from jax import lax
from jax.experimental import pallas as pl
from jax.experimental.pallas import tpu as pltpu
from jax.experimental.pallas import tpu_sc as plsc

_K = 15
_DRUG = 5000
_N = 10000
_D = 128
_E = 320000

_NC = 2
_NS = 16
_FH = _D // _NC
_FV = _FH // 16
_RT = _N // _NS
_ZR = 25
_G = 64
_C = 316
_EPT = _C * _G
_EP = _EPT * _NS
_FR = _RT // _G
_FREM = _RT - _FR * _G


def _frac_coeffs(log_alpha, log_t):
    alpha = jnp.clip(jnp.logaddexp(log_alpha, 0.0), 0.05, 3.0)
    t = jnp.clip(jnp.logaddexp(log_t, 0.0), 0.01, 10.0)
    s = t / (1.0 + t)
    base = (1.0 + t) ** (-alpha)
    coeffs = [base]
    rising = jnp.ones_like(alpha)
    s_pow = jnp.ones_like(alpha)
    for k in range(1, _K + 1):
        rising = rising * (alpha + k - 1) / k
        s_pow = s_pow * s
        coeffs.append(base * rising * s_pow)
    return jnp.stack(coeffs)


def _body(x_hbm, e_hbm, v_hbm, coeff_hbm, out_hbm,
          P, Q, eb0, eb1, eb2, eb3, vb0, vb1, vb2, vb3, gb0, gb1,
          out_t, zbuf, coeff_t,
          es0, es1, es2, es3, vs0, vs1, vs2, vs3, gs0, gs1, ss0, ss1):
    c = lax.axis_index("c")
    s = lax.axis_index("s")
    fbase = c * _FH
    rbase = s * _RT
    ebufs = (eb0, eb1, eb2, eb3)
    vbufs = (vb0, vb1, vb2, vb3)
    gbufs = (gb0, gb1)
    esems = (es0, es1, es2, es3)
    vsems = (vs0, vs1, vs2, vs3)
    gsems = (gs0, gs1)
    ssems = (ss0, ss1)

    crow = jnp.where(s < _DRUG // _RT, 0, 1)
    pltpu.sync_copy(coeff_hbm.at[crow], coeff_t)
    pltpu.sync_copy(x_hbm.at[pl.ds(rbase, _RT), pl.ds(fbase, _FH)],
                    P.at[pl.ds(rbase, _RT)])

    zero16 = jnp.zeros((16,), jnp.float32)

    def _zrow(r, _):
        for f in range(_FV):
            zbuf[r, pl.ds(f * 16, 16)] = zero16
        return 0

    lax.fori_loop(0, _ZR, _zrow, 0)

    def _zout(r, _):
        for f in range(_FV):
            out_t[r, pl.ds(f * 16, 16)] = zero16
        return 0

    lax.fori_loop(0, _RT, _zout, 0)
    for j in range(_RT // _ZR):
        pltpu.sync_copy(zbuf, Q.at[pl.ds(rbase + j * _ZR, _ZR)])
    plsc.subcore_barrier()

    def fma_rows(src_sh, base, rows, obase, ckv):
        pltpu.sync_copy(src_sh.at[pl.ds(base, rows)],
                        gb0.at[pl.ds(0, rows)])

        def rloop(r, _):
            for f in range(_FV):
                sl = pl.ds(f * 16, 16)
                out_t[obase + r, sl] = out_t[obase + r, sl] + ckv * gb0[r, sl]
            return 0

        lax.fori_loop(0, rows, rloop, 0)

    def fma_round(src_sh, kidx):
        ckv = coeff_t[kidx]

        def chunk(j, _):
            fma_rows(src_sh, rbase + j * _G, _G, j * _G, ckv)
            return 0

        lax.fori_loop(0, _FR, chunk, 0)
        fma_rows(src_sh, rbase + _FR * _G, _FREM, _FR * _G, ckv)

    def spmm_round(A, B):
        def fetch(j, q):
            pltpu.async_copy(e_hbm.at[s, j], ebufs[q], esems[q])
            pltpu.async_copy(v_hbm.at[s, j], vbufs[q], vsems[q])

        def ewait(j, q):
            pltpu.make_async_copy(e_hbm.at[s, j], ebufs[q], esems[q]).wait()

        def vwait(j, q):
            pltpu.make_async_copy(v_hbm.at[s, j], vbufs[q], vsems[q]).wait()

        def gissue(q, p):
            pltpu.async_copy(A.at[ebufs[q].at[0]], gbufs[p], gsems[p])

        def gwait(q, p):
            pltpu.make_async_copy(A.at[ebufs[q].at[0]], gbufs[p],
                                  gsems[p]).wait()

        def sissue(q, p):
            pltpu.async_copy(gbufs[p], B.at[ebufs[q].at[1]], ssems[p],
                             add=True)

        def swait(p):
            pltpu.make_async_copy(gbufs[p], B.at[ebufs[0].at[1]],
                                  ssems[p]).wait()

        def scale(q, p):
            gbuf, vbuf = gbufs[p], vbufs[q]

            def scale16(g, _):
                vvec = vbuf[pl.ds(g * 16, 16)]
                for u in range(16):
                    e = g * 16 + u
                    vv = jnp.full((16,), vvec[u], jnp.float32)
                    for f in range(_FV):
                        sl = pl.ds(f * 16, 16)
                        gbuf[e, sl] = gbuf[e, sl] * vv
                return 0

            lax.fori_loop(0, _G // 16, scale16, 0)

        def step(j, u, first=False, do_ga=True, do_fetch=True):
            q, p = u % 4, u % 2
            q1, q3, p1 = (u + 1) % 4, (u + 3) % 4, 1 - (u % 2)
            gwait(q, p)
            if not first:
                swait(p1)
            if do_ga:
                ewait(j + 1, q1)
                gissue(q1, p1)
            if do_fetch:
                fetch(j + 3, q3)
            vwait(j, q)
            scale(q, p)
            sissue(q, p)

        for q in range(3):
            fetch(q, q)
        ewait(0, 0)
        gissue(0, 0)
        for u in range(4):
            step(jnp.int32(u), u, first=(u == 0))

        def body4(i, _):
            jb = 4 * i
            for u in range(4):
                step(jb + u, u)
            return 0

        lax.fori_loop(1, _C // 4 - 1, body4, 0)
        jt = _C - 4
        step(jnp.int32(jt), 0, do_fetch=True)
        step(jnp.int32(jt + 1), 1, do_fetch=False)
        step(jnp.int32(jt + 2), 2, do_fetch=False)
        step(jnp.int32(jt + 3), 3, do_ga=False, do_fetch=False)
        swait(1)

    def finish_round(A):
        plsc.subcore_barrier()
        for j in range(_RT // _ZR):
            pltpu.sync_copy(zbuf, A.at[pl.ds(rbase + j * _ZR, _ZR)])
        plsc.subcore_barrier()

    def pair(i, _):
        fma_round(P, 2 * i)
        spmm_round(P, Q)
        finish_round(P)
        fma_round(Q, 2 * i + 1)
        spmm_round(Q, P)
        finish_round(Q)
        return 0

    lax.fori_loop(0, (_K - 1) // 2, pair, 0)
    fma_round(P, jnp.int32(_K - 1))
    spmm_round(P, Q)
    plsc.subcore_barrier()
    fma_round(Q, jnp.int32(_K))

    pltpu.sync_copy(out_t,
                    out_hbm.at[pl.ds(rbase, _RT), pl.ds(fbase, _FH)])


def kernel(x, edge_index, edge_vals, log_alpha_drug, log_alpha_prot,
           log_t_drug, log_t_prot):
    coeffs = jnp.stack([
        _frac_coeffs(log_alpha_drug, log_t_drug),
        _frac_coeffs(log_alpha_prot, log_t_prot),
    ]).astype(jnp.float32)
    coeffs = jnp.tile(coeffs[:, :, None], (1, 1, 16))

    dst = edge_index[0].astype(jnp.int32)
    src = edge_index[1].astype(jnp.int32)
    vals = edge_vals.astype(jnp.float32)
    pad = _EP - _E
    zi = jnp.zeros((pad,), jnp.int32)
    srcp = jnp.concatenate([src, zi]).reshape(_NS, _C, _G)
    dstp = jnp.concatenate([dst, zi]).reshape(_NS, _C, _G)
    valsp = jnp.concatenate([vals, jnp.zeros((pad,), jnp.float32)]
                            ).reshape(_NS, _C, _G)
    edges = jnp.stack([srcp, dstp], axis=2)

    mesh = plsc.VectorSubcoreMesh(core_axis_name="c", subcore_axis_name="s",
                                  num_cores=_NC, num_subcores=_NS)
    dma = pltpu.SemaphoreType.DMA
    run = functools.partial(
        pl.kernel,
        out_type=jax.ShapeDtypeStruct((_N, _D), jnp.float32),
        mesh=mesh,
        compiler_params=pltpu.CompilerParams(use_tc_tiling_on_sc=False),
        scratch_types=[
            pltpu.VMEM_SHARED((_N, _FH), jnp.float32),
            pltpu.VMEM_SHARED((_N, _FH), jnp.float32),
            pltpu.VMEM((2, _G), jnp.int32),
            pltpu.VMEM((2, _G), jnp.int32),
            pltpu.VMEM((2, _G), jnp.int32),
            pltpu.VMEM((2, _G), jnp.int32),
            pltpu.VMEM((_G,), jnp.float32),
            pltpu.VMEM((_G,), jnp.float32),
            pltpu.VMEM((_G,), jnp.float32),
            pltpu.VMEM((_G,), jnp.float32),
            pltpu.VMEM((_G, _FH), jnp.float32),
            pltpu.VMEM((_G, _FH), jnp.float32),
            pltpu.VMEM((_RT, _FH), jnp.float32),
            pltpu.VMEM((_ZR, _FH), jnp.float32),
            pltpu.VMEM((16, 16), jnp.float32),
            dma, dma, dma, dma,
            dma, dma, dma, dma,
            dma, dma,
            dma, dma,
        ],
    )(_body)
    return run(x, edges, valsp, coeffs)
